# baseline (speedup 1.0000x reference)
import functools

import jax
import jax.numpy as jnp
from jax import lax
from jax.experimental import pallas as pl
from jax.experimental.pallas import tpu as pltpu
from jax.experimental.pallas import tpu_sc as plsc

N = 100000
D = 64
B = 16384
L = 16
NC = 2
NS = 16
NW = NC * NS
BPW = B // NW
NVEC = B // L
MCHUNK = 128

_mesh = plsc.VectorSubcoreMesh(core_axis_name="c", subcore_axis_name="s")


@functools.partial(
    pl.kernel,
    out_type=tuple(jax.ShapeDtypeStruct((B, D), jnp.float32) for _ in range(5)),
    mesh=_mesh,
    compiler_params=pltpu.CompilerParams(needs_layout_passes=False),
    scratch_types=[
        pltpu.VMEM((N,), jnp.int32),
        pltpu.VMEM((B // 4,), jnp.int32),
        pltpu.VMEM((BPW // MCHUNK, MCHUNK), jnp.int32),
        pltpu.VMEM((BPW,), jnp.int32),
        pltpu.VMEM_SHARED((N,), jnp.int32),
        pltpu.SemaphoreType.DMA,
        pltpu.SemaphoreType.DMA,
    ],
)
def _memory_kernel(idx_hbm, zeros_hbm, v1, v2, v3, v4, v5,
                   o1, o2, o3, o4, o5,
                   pos, xm, idx_v, m_v, pos_sh, msem, rsem):
    core = lax.axis_index("c")
    sub = lax.axis_index("s")
    wid = sub * NC + core
    base = wid * BPW

    @pl.when(sub == 0)
    def _():
        pltpu.sync_copy(zeros_hbm, pos)
        lane = lax.iota(jnp.int32, L)
        seg = B // 4
        seg_vecs = seg // L

        for g in range(4):
            pltpu.sync_copy(idx_hbm.at[pl.ds(g * seg, seg)], xm)

            def scatter_body(c, carry, g=g):
                x = xm[pl.ds(c * L, L)]
                j = lane + (g * seg_vecs + c) * L
                plsc.store_scatter(pos, [x], j)
                r = plsc.load_gather(pos, [x])
                n0 = plsc.all_reduce_population_count(j > r)[0]

                def retry_cond(st):
                    return st[0] > 0

                def retry_body(st):
                    plsc.store_scatter(pos, [x], j, mask=j > st[1])
                    r2 = plsc.load_gather(pos, [x])
                    return (plsc.all_reduce_population_count(j > r2)[0], r2)

                lax.while_loop(retry_cond, retry_body, (n0, r))
                return carry

            lax.fori_loop(0, seg_vecs, scatter_body, 0)
        pltpu.sync_copy(pos, pos_sh)

    plsc.subcore_barrier()

    for j in range(BPW // MCHUNK):
        pltpu.sync_copy(idx_hbm.at[pl.ds(base + j * MCHUNK, MCHUNK)],
                        idx_v.at[j])
    mcp = [pltpu.async_copy(pos_sh.at[idx_v.at[j]],
                            m_v.at[pl.ds(j * MCHUNK, MCHUNK)], msem)
           for j in range(BPW // MCHUNK)]
    for cp in mcp:
        cp.wait()

    vs = (v1, v2, v3, v4, v5)
    os_ = (o1, o2, o3, o4, o5)

    def group_body(g, carry):
        mvec = m_v[pl.ds(g * L, L)]
        row0 = base + g * L
        for l in range(L):
            mi = mvec[l]
            for k in range(5):
                pltpu.async_copy(vs[k].at[pl.ds(mi, 1)],
                                 os_[k].at[pl.ds(row0 + l, 1)], rsem)
        return carry

    lax.fori_loop(0, BPW // L, group_body, 0)

    for k in range(5):
        pltpu.make_async_copy(
            os_[k].at[pl.ds(base, BPW)], os_[k].at[pl.ds(base, BPW)],
            rsem).wait()


def kernel(nodes_memory, crowds_memory, interests_memory, categories_memory,
           brands_memory, values1, values2, values3, values4, values5,
           users_idxs):
    zeros = jnp.zeros((N,), jnp.int32)
    return _memory_kernel(users_idxs, zeros, values1, values2, values3,
                          values4, values5)


# speedup vs baseline: 6.7623x; 6.7623x over previous
import functools

import jax
import jax.numpy as jnp
from jax import lax
from jax.experimental import pallas as pl
from jax.experimental.pallas import tpu as pltpu
from jax.experimental.pallas import tpu_sc as plsc

N = 100000
D = 64
B = 16384
L = 16
NC = 2
NS = 16
NW = NC * NS
BPW = B // NW
CHUNK = 128
NCHUNK = BPW // CHUNK

_mesh = plsc.VectorSubcoreMesh(core_axis_name="c", subcore_axis_name="s")


@functools.partial(
    pl.kernel,
    out_type=jax.ShapeDtypeStruct((B,), jnp.int32),
    mesh=_mesh,
    compiler_params=pltpu.CompilerParams(needs_layout_passes=False),
    scratch_types=[
        pltpu.VMEM((N,), jnp.int32),
        pltpu.VMEM((B,), jnp.int32),
    ],
)
def _last_writer(idx_hbm, zeros_hbm, m_hbm, pos, xm):
    core = lax.axis_index("c")
    sub = lax.axis_index("s")

    @pl.when(jnp.logical_and(core == 0, sub == 0))
    def _():
        pltpu.sync_copy(zeros_hbm, pos)
        pltpu.sync_copy(idx_hbm, xm)
        lane = lax.iota(jnp.int32, L)

        def scatter_body(c, carry):
            x = xm[pl.ds(c * L, L)]
            j = lane + c * L
            plsc.store_scatter(pos, [x], j)
            r = plsc.load_gather(pos, [x])
            n0 = plsc.all_reduce_population_count(j > r)[0]

            def retry_cond(st):
                return st[0] > 0

            def retry_body(st):
                plsc.store_scatter(pos, [x], j, mask=j > st[1])
                r2 = plsc.load_gather(pos, [x])
                return (plsc.all_reduce_population_count(j > r2)[0], r2)

            lax.while_loop(retry_cond, retry_body, (n0, r))
            return carry

        lax.fori_loop(0, B // L, scatter_body, 0)

        def gather_body(c, carry):
            x = xm[pl.ds(c * L, L)]
            xm[pl.ds(c * L, L)] = plsc.load_gather(pos, [x])
            return carry

        lax.fori_loop(0, B // L, gather_body, 0)
        pltpu.sync_copy(xm, m_hbm)


@functools.partial(
    pl.kernel,
    out_type=jax.ShapeDtypeStruct((B, D), jnp.float32),
    mesh=_mesh,
    compiler_params=pltpu.CompilerParams(
        needs_layout_passes=False, use_tc_tiling_on_sc=False),
    scratch_types=[
        pltpu.VMEM((NCHUNK, CHUNK), jnp.int32),
        pltpu.VMEM((CHUNK, D), jnp.float32),
        pltpu.VMEM((CHUNK, D), jnp.float32),
        pltpu.SemaphoreType.DMA,
        pltpu.SemaphoreType.DMA,
        pltpu.SemaphoreType.DMA,
        pltpu.SemaphoreType.DMA,
    ],
)
def _gather_rows(m_hbm, v, o, m_v, buf_a, buf_b,
                 gsem_a, gsem_b, wsem_a, wsem_b):
    core = lax.axis_index("c")
    sub = lax.axis_index("s")
    wid = sub * NC + core
    base = wid * BPW

    for j in range(NCHUNK):
        pltpu.sync_copy(m_hbm.at[pl.ds(base + j * CHUNK, CHUNK)], m_v.at[j])

    bufs = (buf_a, buf_b)
    gsems = (gsem_a, gsem_b)
    wsems = (wsem_a, wsem_b)

    def fire(j):
        return pltpu.async_copy(v.at[m_v.at[j]], bufs[j % 2], gsems[j % 2])

    wcp = [None, None]
    cp = fire(0)
    for j in range(NCHUNK):
        if j + 1 < NCHUNK:
            if wcp[(j + 1) % 2] is not None:
                wcp[(j + 1) % 2].wait()
            nxt = fire(j + 1)
        else:
            nxt = None
        cp.wait()
        wcp[j % 2] = pltpu.async_copy(
            bufs[j % 2], o.at[pl.ds(base + j * CHUNK, CHUNK)], wsems[j % 2])
        cp = nxt
    for w in wcp:
        if w is not None:
            w.wait()


def kernel(nodes_memory, crowds_memory, interests_memory, categories_memory,
           brands_memory, values1, values2, values3, values4, values5,
           users_idxs):
    zeros = jnp.zeros((N,), jnp.int32)
    m = _last_writer(users_idxs, zeros)
    return tuple(_gather_rows(m, v)
                 for v in (values1, values2, values3, values4, values5))
